# Initial kernel scaffold; baseline (speedup 1.0000x reference)
import jax
import jax.numpy as jnp
from jax.experimental import pallas as pl


def kernel(x, edge_index, batch, W1, b1, W2, b2):
    raise NotImplementedError("write your pallas kernel here")


# speedup vs baseline: 25.1993x; 25.1993x over previous
import functools

import jax
import jax.numpy as jnp
from jax import lax
from jax.experimental import pallas as pl
from jax.experimental.pallas import tpu as pltpu
from jax.experimental.pallas import tpu_sc as plsc

_NC = 2
_NS = 16
_NW = _NC * _NS

_CH = 80


def _mesh():
    return plsc.VectorSubcoreMesh(
        core_axis_name="c", subcore_axis_name="s", num_cores=_NC, num_subcores=_NS
    )


def _part(n):
    step = (n // _NS) & ~7
    size = n - step * (_NS - 1)
    assert size % 8 == 0 and size >= step
    return step, size


def _make_deg_kernel(n, nch):
    step, size = _part(n)

    def body(col_hbm, zeros_hbm, out_hbm, col_v, ones_v, acc, _sem):
        c = lax.axis_index("c")
        s = lax.axis_index("s")
        wid = s * _NC + c
        off = s * step
        pltpu.sync_copy(col_hbm.at[wid], col_v)
        pltpu.sync_copy(
            zeros_hbm.at[pl.ds(off, size)],
            acc.at[pl.ds(off, size)],
        )
        ones16 = jnp.ones((16,), jnp.float32)

        def fill(i, carry):
            ones_v[i] = ones16
            return carry

        lax.fori_loop(0, _CH, fill, 0)
        plsc.subcore_barrier()

        def chunk(j, carry):
            pltpu.sync_copy(ones_v, acc.at[col_v.at[j]], add=True)
            return carry

        lax.fori_loop(0, nch, chunk, 0)
        plsc.subcore_barrier()
        pltpu.sync_copy(
            acc.at[pl.ds(off, size)],
            out_hbm.at[c, pl.ds(off, size)],
        )

    return pl.kernel(
        body,
        out_type=jax.ShapeDtypeStruct((_NC, n, 16), jnp.float32),
        mesh=_mesh(),
        compiler_params=pltpu.CompilerParams(use_tc_tiling_on_sc=False),
        scratch_types=[
            pltpu.VMEM((nch, _CH), jnp.int32),
            pltpu.VMEM((_CH, 16), jnp.float32),
            pltpu.VMEM_SHARED((n, 16), jnp.float32),
            pltpu.SemaphoreType.DMA,
        ],
    )


def _make_mp_kernel(n, d, nch):
    step, size = _part(n)

    def body(y_hbm, row_hbm, col_hbm, zeros_hbm, out_hbm, row_v, col_v, buf, acc, sem):
        c = lax.axis_index("c")
        s = lax.axis_index("s")
        wid = s * _NC + c
        off = s * step
        pltpu.sync_copy(row_hbm.at[wid], row_v)
        pltpu.sync_copy(col_hbm.at[wid], col_v)
        pltpu.sync_copy(
            zeros_hbm.at[pl.ds(off, size)],
            acc.at[pl.ds(off, size)],
        )
        plsc.subcore_barrier()

        def chunk(j, carry):
            pltpu.async_copy(y_hbm.at[row_v.at[j]], buf, sem).wait()
            pltpu.sync_copy(buf, acc.at[col_v.at[j]], add=True)
            return carry

        lax.fori_loop(0, nch, chunk, 0)
        plsc.subcore_barrier()
        pltpu.sync_copy(
            acc.at[pl.ds(off, size)],
            out_hbm.at[c, pl.ds(off, size)],
        )

    return pl.kernel(
        body,
        out_type=jax.ShapeDtypeStruct((_NC, n, d), jnp.float32),
        mesh=_mesh(),
        compiler_params=pltpu.CompilerParams(use_tc_tiling_on_sc=False),
        scratch_types=[
            pltpu.VMEM((nch, _CH), jnp.int32),
            pltpu.VMEM((nch, _CH), jnp.int32),
            pltpu.VMEM((_CH, d), jnp.float32),
            pltpu.VMEM_SHARED((n, d), jnp.float32),
            pltpu.SemaphoreType.DMA,
        ],
    )


def _tc1_body(x_ref, w1_ref, degp_ref, dinv_ref, y1_ref):
    deg = degp_ref[0, :, 0] + degp_ref[1, :, 0] + 1.0
    dinv = lax.rsqrt(deg)[:, None]
    dinv_ref[...] = dinv
    xw = jnp.dot(x_ref[...], w1_ref[...], preferred_element_type=jnp.float32)
    y1_ref[...] = dinv * xw


def _tc2_body(t1_ref, y1_ref, dinv_ref, b1_ref, w2_ref, y2_ref):
    dinv = dinv_ref[...]
    h1 = jax.nn.relu(dinv * (t1_ref[0] + t1_ref[1] + y1_ref[...]) + b1_ref[...])
    y2_ref[...] = dinv * jnp.dot(h1, w2_ref[...], preferred_element_type=jnp.float32)


def _tc3_body(t2_ref, y2_ref, dinv_ref, b2_ref, batch_ref, out_ref, *, g):
    dinv = dinv_ref[...]
    h2 = jax.nn.relu(dinv * (t2_ref[0] + t2_ref[1] + y2_ref[...]) + b2_ref[...])
    n = h2.shape[0]
    seg = lax.broadcasted_iota(jnp.int32, (g, n), 0)
    mask = (seg == batch_ref[...]).astype(jnp.float32)
    sums = jnp.dot(mask, h2, preferred_element_type=jnp.float32)
    cnt = jnp.sum(mask, axis=1, keepdims=True)
    out_ref[...] = sums / jnp.maximum(cnt, 1.0)


def kernel(x, edge_index, batch, W1, b1, W2, b2):
    n, _ = x.shape
    e = edge_index.shape[1]
    d = W1.shape[1]
    g = 64

    ew = e // _NW
    nch = ew // _CH
    row_r = edge_index[0].reshape(_NW, nch, _CH).astype(jnp.int32)
    col_r = edge_index[1].reshape(_NW, nch, _CH).astype(jnp.int32)

    z16 = jnp.zeros((n, 16), jnp.float32)
    zd = jnp.zeros((n, d), jnp.float32)

    degp = _make_deg_kernel(n, nch)(col_r, z16)

    dinv, y1 = pl.pallas_call(
        _tc1_body,
        out_shape=[
            jax.ShapeDtypeStruct((n, 1), jnp.float32),
            jax.ShapeDtypeStruct((n, d), jnp.float32),
        ],
    )(x, W1, degp)

    mp = _make_mp_kernel(n, d, nch)
    t1 = mp(y1, row_r, col_r, zd)

    y2 = pl.pallas_call(
        _tc2_body,
        out_shape=jax.ShapeDtypeStruct((n, d), jnp.float32),
    )(t1, y1, dinv, b1.reshape(1, d), W2)

    t2 = mp(y2, row_r, col_r, zd)

    out = pl.pallas_call(
        functools.partial(_tc3_body, g=g),
        out_shape=jax.ShapeDtypeStruct((g, d), jnp.float32),
    )(t2, y2, dinv, b2.reshape(1, d), batch.reshape(1, n).astype(jnp.int32))
    return out


# speedup vs baseline: 36.6619x; 1.4549x over previous
import functools

import jax
import jax.numpy as jnp
from jax import lax
from jax.experimental import pallas as pl
from jax.experimental.pallas import tpu as pltpu
from jax.experimental.pallas import tpu_sc as plsc

_NC = 2
_NS = 16
_NW = _NC * _NS

_CH = 80


def _mesh():
    return plsc.VectorSubcoreMesh(
        core_axis_name="c", subcore_axis_name="s", num_cores=_NC, num_subcores=_NS
    )


def _part(n):
    step = (n // _NS) & ~7
    size = n - step * (_NS - 1)
    assert size % 8 == 0 and size >= step
    return step, size


def _make_deg_kernel(n, nch):
    step, size = _part(n)

    def body(col_hbm, zeros_hbm, out_hbm, col_v, ones_v, acc, _sem):
        c = lax.axis_index("c")
        s = lax.axis_index("s")
        wid = s * _NC + c
        off = s * step
        pltpu.sync_copy(col_hbm.at[wid], col_v)
        pltpu.sync_copy(
            zeros_hbm.at[pl.ds(off, size)],
            acc.at[pl.ds(off, size)],
        )
        ones16 = jnp.ones((16,), jnp.float32)

        def fill(i, carry):
            ones_v[i] = ones16
            return carry

        lax.fori_loop(0, _CH, fill, 0)
        plsc.subcore_barrier()

        def chunk(j, carry):
            pltpu.sync_copy(ones_v, acc.at[col_v.at[j]], add=True)
            return carry

        lax.fori_loop(0, nch, chunk, 0)
        plsc.subcore_barrier()
        pltpu.sync_copy(
            acc.at[pl.ds(off, size)],
            out_hbm.at[c, pl.ds(off, size)],
        )

    return pl.kernel(
        body,
        out_type=jax.ShapeDtypeStruct((_NC, n, 16), jnp.float32),
        mesh=_mesh(),
        compiler_params=pltpu.CompilerParams(use_tc_tiling_on_sc=False),
        scratch_types=[
            pltpu.VMEM((nch, _CH), jnp.int32),
            pltpu.VMEM((_CH, 16), jnp.float32),
            pltpu.VMEM_SHARED((n, 16), jnp.float32),
            pltpu.SemaphoreType.DMA,
        ],
    )


def _make_mp_kernel(n, d, nch):
    step, size = _part(n)

    assert nch % 2 == 1

    def body(
        y_hbm, row_hbm, col_hbm, zeros_hbm, out_hbm,
        row_v, col_v, buf0, buf1, acc, sem0, sem1,
    ):
        c = lax.axis_index("c")
        s = lax.axis_index("s")
        wid = s * _NC + c
        off = s * step
        pltpu.sync_copy(row_hbm.at[wid], row_v)
        pltpu.sync_copy(col_hbm.at[wid], col_v)
        pltpu.sync_copy(
            zeros_hbm.at[pl.ds(off, size)],
            acc.at[pl.ds(off, size)],
        )
        plsc.subcore_barrier()

        pltpu.async_copy(y_hbm.at[row_v.at[0]], buf0, sem0)

        def pair(i, carry):
            j0 = 2 * i
            pltpu.async_copy(y_hbm.at[row_v.at[j0 + 1]], buf1, sem1)
            pltpu.make_async_copy(y_hbm.at[row_v.at[j0]], buf0, sem0).wait()
            pltpu.sync_copy(buf0, acc.at[col_v.at[j0]], add=True)
            pltpu.async_copy(y_hbm.at[row_v.at[j0 + 2]], buf0, sem0)
            pltpu.make_async_copy(y_hbm.at[row_v.at[j0 + 1]], buf1, sem1).wait()
            pltpu.sync_copy(buf1, acc.at[col_v.at[j0 + 1]], add=True)
            return carry

        lax.fori_loop(0, nch // 2, pair, 0)
        pltpu.make_async_copy(y_hbm.at[row_v.at[nch - 1]], buf0, sem0).wait()
        pltpu.sync_copy(buf0, acc.at[col_v.at[nch - 1]], add=True)
        plsc.subcore_barrier()
        pltpu.sync_copy(
            acc.at[pl.ds(off, size)],
            out_hbm.at[c, pl.ds(off, size)],
        )

    return pl.kernel(
        body,
        out_type=jax.ShapeDtypeStruct((_NC, n, d), jnp.float32),
        mesh=_mesh(),
        compiler_params=pltpu.CompilerParams(use_tc_tiling_on_sc=False),
        scratch_types=[
            pltpu.VMEM((nch, _CH), jnp.int32),
            pltpu.VMEM((nch, _CH), jnp.int32),
            pltpu.VMEM((_CH, d), jnp.float32),
            pltpu.VMEM((_CH, d), jnp.float32),
            pltpu.VMEM_SHARED((n, d), jnp.float32),
            pltpu.SemaphoreType.DMA,
            pltpu.SemaphoreType.DMA,
        ],
    )


def _tc1_body(x_ref, w1_ref, degp_ref, dinv_ref, y1_ref):
    deg = degp_ref[0, :, 0] + degp_ref[1, :, 0] + 1.0
    dinv = lax.rsqrt(deg)[:, None]
    dinv_ref[...] = dinv
    xw = jnp.dot(x_ref[...], w1_ref[...], preferred_element_type=jnp.float32)
    y1_ref[...] = dinv * xw


def _tc2_body(t1_ref, y1_ref, dinv_ref, b1_ref, w2_ref, y2_ref):
    dinv = dinv_ref[...]
    h1 = jax.nn.relu(dinv * (t1_ref[0] + t1_ref[1] + y1_ref[...]) + b1_ref[...])
    y2_ref[...] = dinv * jnp.dot(h1, w2_ref[...], preferred_element_type=jnp.float32)


def _tc3_body(t2_ref, y2_ref, dinv_ref, b2_ref, batch_ref, out_ref, *, g):
    dinv = dinv_ref[...]
    h2 = jax.nn.relu(dinv * (t2_ref[0] + t2_ref[1] + y2_ref[...]) + b2_ref[...])
    n = h2.shape[0]
    seg = lax.broadcasted_iota(jnp.int32, (g, n), 0)
    mask = (seg == batch_ref[...]).astype(jnp.float32)
    sums = jnp.dot(mask, h2, preferred_element_type=jnp.float32)
    cnt = jnp.sum(mask, axis=1, keepdims=True)
    out_ref[...] = sums / jnp.maximum(cnt, 1.0)


def kernel(x, edge_index, batch, W1, b1, W2, b2):
    n, _ = x.shape
    e = edge_index.shape[1]
    d = W1.shape[1]
    g = 64

    ew = e // _NW
    nch = ew // _CH
    row_r = edge_index[0].reshape(_NW, nch, _CH).astype(jnp.int32)
    col_r = edge_index[1].reshape(_NW, nch, _CH).astype(jnp.int32)

    z16 = jnp.zeros((n, 16), jnp.float32)
    zd = jnp.zeros((n, d), jnp.float32)

    degp = _make_deg_kernel(n, nch)(col_r, z16)

    dinv, y1 = pl.pallas_call(
        _tc1_body,
        out_shape=[
            jax.ShapeDtypeStruct((n, 1), jnp.float32),
            jax.ShapeDtypeStruct((n, d), jnp.float32),
        ],
    )(x, W1, degp)

    mp = _make_mp_kernel(n, d, nch)
    t1 = mp(y1, row_r, col_r, zd)

    y2 = pl.pallas_call(
        _tc2_body,
        out_shape=jax.ShapeDtypeStruct((n, d), jnp.float32),
    )(t1, y1, dinv, b1.reshape(1, d), W2)

    t2 = mp(y2, row_r, col_r, zd)

    out = pl.pallas_call(
        functools.partial(_tc3_body, g=g),
        out_shape=jax.ShapeDtypeStruct((g, d), jnp.float32),
    )(t2, y2, dinv, b2.reshape(1, d), batch.reshape(1, n).astype(jnp.int32))
    return out


# speedup vs baseline: 46.6528x; 1.2725x over previous
import functools

import jax
import jax.numpy as jnp
from jax import lax
from jax.experimental import pallas as pl
from jax.experimental.pallas import tpu as pltpu
from jax.experimental.pallas import tpu_sc as plsc

_NC = 2
_NS = 16
_NW = _NC * _NS

_CH = 80


def _mesh():
    return plsc.VectorSubcoreMesh(
        core_axis_name="c", subcore_axis_name="s", num_cores=_NC, num_subcores=_NS
    )


def _part(n):
    step = (n // _NS) & ~7
    size = n - step * (_NS - 1)
    assert size % 8 == 0 and size >= step
    return step, size


def _make_deg_kernel(n, nch):
    step, size = _part(n)

    def body(edge_hbm, zeros_hbm, out_hbm, col_v, ones_v, acc, _sem):
        c = lax.axis_index("c")
        s = lax.axis_index("s")
        wid = s * _NC + c
        off = s * step
        pltpu.sync_copy(edge_hbm.at[1, wid], col_v)
        pltpu.sync_copy(
            zeros_hbm.at[pl.ds(off, size)],
            acc.at[pl.ds(off, size)],
        )
        ones16 = jnp.ones((16,), jnp.float32)

        def fill(i, carry):
            ones_v[i] = ones16
            return carry

        lax.fori_loop(0, _CH, fill, 0)
        plsc.subcore_barrier()

        def chunk(j, carry):
            pltpu.sync_copy(ones_v, acc.at[col_v.at[j]], add=True)
            return carry

        lax.fori_loop(0, nch, chunk, 0)
        plsc.subcore_barrier()
        pltpu.sync_copy(
            acc.at[pl.ds(off, size)],
            out_hbm.at[c, pl.ds(off, size)],
        )

    return pl.kernel(
        body,
        out_type=jax.ShapeDtypeStruct((_NC, n, 16), jnp.float32),
        mesh=_mesh(),
        compiler_params=pltpu.CompilerParams(use_tc_tiling_on_sc=False),
        scratch_types=[
            pltpu.VMEM((nch, _CH), jnp.int32),
            pltpu.VMEM((_CH, 16), jnp.float32),
            pltpu.VMEM_SHARED((n, 16), jnp.float32),
            pltpu.SemaphoreType.DMA,
        ],
    )


def _make_mp_kernel(n, d, nch):
    step, size = _part(n)

    _NB = 8
    _PF = 4

    def body(
        y_hbm, edge_hbm, zeros_hbm, out_hbm,
        row_v, col_v, buf, acc, gsem, ssem,
    ):
        c = lax.axis_index("c")
        s = lax.axis_index("s")
        wid = s * _NC + c
        off = s * step
        pltpu.sync_copy(edge_hbm.at[0, wid], row_v)
        pltpu.sync_copy(edge_hbm.at[1, wid], col_v)

        def gather(j, p):
            return pltpu.async_copy(y_hbm.at[row_v.at[j]], buf.at[p], gsem.at[p])

        def scatter(j, p):
            return pltpu.async_copy(
                buf.at[p], acc.at[col_v.at[j]], ssem.at[p], add=True
            )

        for j in range(_PF):
            gather(j, j)
        pltpu.sync_copy(
            zeros_hbm.at[pl.ds(off, size)],
            acc.at[pl.ds(off, size)],
        )
        plsc.subcore_barrier()

        def step_fn(j, carry):
            p = lax.rem(j, _NB)
            q = lax.rem(j + _PF, _NB)
            pltpu.make_async_copy(y_hbm.at[row_v.at[j]], buf.at[p], gsem.at[p]).wait()
            scatter(j, p)

            @pl.when(j >= _PF)
            def _():
                jq = j - _PF
                pltpu.make_async_copy(
                    buf.at[q], acc.at[col_v.at[jq]], ssem.at[q]
                ).wait()

            @pl.when(j + _PF < nch)
            def _():
                gather(j + _PF, q)

            return carry

        lax.fori_loop(0, nch, step_fn, 0)
        for jj in range(nch - _PF, nch):
            p = jj % _NB
            pltpu.make_async_copy(buf.at[p], acc.at[col_v.at[jj]], ssem.at[p]).wait()
        plsc.subcore_barrier()
        pltpu.sync_copy(
            acc.at[pl.ds(off, size)],
            out_hbm.at[c, pl.ds(off, size)],
        )

    return pl.kernel(
        body,
        out_type=jax.ShapeDtypeStruct((_NC, n, d), jnp.float32),
        mesh=_mesh(),
        compiler_params=pltpu.CompilerParams(use_tc_tiling_on_sc=False),
        scratch_types=[
            pltpu.VMEM((nch, _CH), jnp.int32),
            pltpu.VMEM((nch, _CH), jnp.int32),
            pltpu.VMEM((_NB, _CH, d), jnp.float32),
            pltpu.VMEM_SHARED((n, d), jnp.float32),
            pltpu.SemaphoreType.DMA((_NB,)),
            pltpu.SemaphoreType.DMA((_NB,)),
        ],
    )


def _tc1_body(x_ref, w1_ref, degp_ref, dinv_ref, y1_ref):
    deg = degp_ref[0, :, 0] + degp_ref[1, :, 0] + 1.0
    dinv = lax.rsqrt(deg)[:, None]
    dinv_ref[...] = dinv
    xw = jnp.dot(x_ref[...], w1_ref[...], preferred_element_type=jnp.float32)
    y1_ref[...] = dinv * xw


def _tc2_body(t1_ref, y1_ref, dinv_ref, b1_ref, w2_ref, y2_ref):
    dinv = dinv_ref[...]
    h1 = jax.nn.relu(dinv * (t1_ref[0] + t1_ref[1] + y1_ref[...]) + b1_ref[...])
    y2_ref[...] = dinv * jnp.dot(h1, w2_ref[...], preferred_element_type=jnp.float32)


def _tc3_body(t2_ref, y2_ref, dinv_ref, b2_ref, batch_ref, out_ref, *, g):
    dinv = dinv_ref[...]
    h2 = jax.nn.relu(dinv * (t2_ref[0] + t2_ref[1] + y2_ref[...]) + b2_ref[...])
    n = h2.shape[0]
    seg = lax.broadcasted_iota(jnp.int32, (g, n), 0)
    mask = (seg == batch_ref[...]).astype(jnp.float32)
    sums = jnp.dot(mask, h2, preferred_element_type=jnp.float32)
    cnt = jnp.sum(mask, axis=1, keepdims=True)
    out_ref[...] = sums / jnp.maximum(cnt, 1.0)


def kernel(x, edge_index, batch, W1, b1, W2, b2):
    n, _ = x.shape
    e = edge_index.shape[1]
    d = W1.shape[1]
    g = 64

    ew = e // _NW
    nch = ew // _CH
    er = edge_index.reshape(2, _NW, nch, _CH)

    z16 = jnp.zeros((n, 16), jnp.float32)
    zd = jnp.zeros((n, d), jnp.float32)

    degp = _make_deg_kernel(n, nch)(er, z16)

    dinv, y1 = pl.pallas_call(
        _tc1_body,
        out_shape=[
            jax.ShapeDtypeStruct((n, 1), jnp.float32),
            jax.ShapeDtypeStruct((n, d), jnp.float32),
        ],
    )(x, W1, degp)

    mp = _make_mp_kernel(n, d, nch)
    t1 = mp(y1, er, zd)

    y2 = pl.pallas_call(
        _tc2_body,
        out_shape=jax.ShapeDtypeStruct((n, d), jnp.float32),
    )(t1, y1, dinv, b1.reshape(1, d), W2)

    t2 = mp(y2, er, zd)

    out = pl.pallas_call(
        functools.partial(_tc3_body, g=g),
        out_shape=jax.ShapeDtypeStruct((g, d), jnp.float32),
    )(t2, y2, dinv, b2.reshape(1, d), batch.reshape(1, n).astype(jnp.int32))
    return out


# speedup vs baseline: 49.8054x; 1.0676x over previous
import functools

import jax
import jax.numpy as jnp
from jax import lax
from jax.experimental import pallas as pl
from jax.experimental.pallas import tpu as pltpu
from jax.experimental.pallas import tpu_sc as plsc

_NC = 2
_NS = 16
_NW = _NC * _NS

_CH = 80


def _mesh():
    return plsc.VectorSubcoreMesh(
        core_axis_name="c", subcore_axis_name="s", num_cores=_NC, num_subcores=_NS
    )


def _part(n):
    step = (n // _NS) & ~7
    size = n - step * (_NS - 1)
    assert size % 8 == 0 and size >= step
    return step, size


def _make_deg_kernel(n, nch):
    step, size = _part(n)

    def body(edge_hbm, zeros_hbm, out_hbm, col_v, ones_v, acc, sems):
        c = lax.axis_index("c")
        s = lax.axis_index("s")
        wid = s * _NC + c
        off = s * step
        pltpu.sync_copy(edge_hbm.at[1, wid], col_v)
        pltpu.sync_copy(
            zeros_hbm.at[pl.ds(off, size)],
            acc.at[pl.ds(off, size)],
        )
        ones16 = jnp.ones((16,), jnp.float32)

        def fill(i, carry):
            ones_v[i] = ones16
            return carry

        lax.fori_loop(0, _CH, fill, 0)
        plsc.subcore_barrier()

        nsem = 8

        def chunk(j, carry):
            p = lax.rem(j, nsem)

            @pl.when(j >= nsem)
            def _():
                pltpu.make_async_copy(
                    ones_v, acc.at[col_v.at[j - nsem]], sems.at[p]
                ).wait()

            pltpu.async_copy(ones_v, acc.at[col_v.at[j]], sems.at[p], add=True)
            return carry

        lax.fori_loop(0, nch, chunk, 0)
        for jj in range(nch - nsem, nch):
            pltpu.make_async_copy(ones_v, acc.at[col_v.at[jj]], sems.at[jj % nsem]).wait()
        plsc.subcore_barrier()
        pltpu.sync_copy(
            acc.at[pl.ds(off, size)],
            out_hbm.at[c, pl.ds(off, size)],
        )

    return pl.kernel(
        body,
        out_type=jax.ShapeDtypeStruct((_NC, n, 16), jnp.float32),
        mesh=_mesh(),
        compiler_params=pltpu.CompilerParams(use_tc_tiling_on_sc=False),
        scratch_types=[
            pltpu.VMEM((nch, _CH), jnp.int32),
            pltpu.VMEM((_CH, 16), jnp.float32),
            pltpu.VMEM_SHARED((n, 16), jnp.float32),
            pltpu.SemaphoreType.DMA((8,)),
        ],
    )


def _make_mp_kernel(n, d, nch):
    step, size = _part(n)

    _NB = 12
    _PF = 6

    def body(
        y_hbm, edge_hbm, zeros_hbm, out_hbm,
        row_v, col_v, buf, acc, gsem, ssem,
    ):
        c = lax.axis_index("c")
        s = lax.axis_index("s")
        wid = s * _NC + c
        off = s * step
        pltpu.sync_copy(edge_hbm.at[0, wid], row_v)
        pltpu.sync_copy(edge_hbm.at[1, wid], col_v)

        def gather(j, p):
            return pltpu.async_copy(y_hbm.at[row_v.at[j]], buf.at[p], gsem.at[p])

        def scatter(j, p):
            return pltpu.async_copy(
                buf.at[p], acc.at[col_v.at[j]], ssem.at[p], add=True
            )

        for j in range(_PF):
            gather(j, j)
        pltpu.sync_copy(
            zeros_hbm.at[pl.ds(off, size)],
            acc.at[pl.ds(off, size)],
        )
        plsc.subcore_barrier()

        def step_fn(j, carry):
            p = lax.rem(j, _NB)
            q = lax.rem(j + _PF, _NB)
            pltpu.make_async_copy(y_hbm.at[row_v.at[j]], buf.at[p], gsem.at[p]).wait()
            scatter(j, p)

            @pl.when(j >= _PF)
            def _():
                jq = j - _PF
                pltpu.make_async_copy(
                    buf.at[q], acc.at[col_v.at[jq]], ssem.at[q]
                ).wait()

            @pl.when(j + _PF < nch)
            def _():
                gather(j + _PF, q)

            return carry

        lax.fori_loop(0, nch, step_fn, 0)
        for jj in range(nch - _PF, nch):
            p = jj % _NB
            pltpu.make_async_copy(buf.at[p], acc.at[col_v.at[jj]], ssem.at[p]).wait()
        plsc.subcore_barrier()
        pltpu.sync_copy(
            acc.at[pl.ds(off, size)],
            out_hbm.at[c, pl.ds(off, size)],
        )

    return pl.kernel(
        body,
        out_type=jax.ShapeDtypeStruct((_NC, n, d), jnp.float32),
        mesh=_mesh(),
        compiler_params=pltpu.CompilerParams(use_tc_tiling_on_sc=False),
        scratch_types=[
            pltpu.VMEM((nch, _CH), jnp.int32),
            pltpu.VMEM((nch, _CH), jnp.int32),
            pltpu.VMEM((_NB, _CH, d), jnp.float32),
            pltpu.VMEM_SHARED((n, d), jnp.float32),
            pltpu.SemaphoreType.DMA((_NB,)),
            pltpu.SemaphoreType.DMA((_NB,)),
        ],
    )


def _tc1_body(x_ref, w1_ref, degp_ref, dinv_ref, y1_ref):
    deg = degp_ref[0, :, 0] + degp_ref[1, :, 0] + 1.0
    dinv = lax.rsqrt(deg)[:, None]
    dinv_ref[...] = dinv
    xw = jnp.dot(x_ref[...], w1_ref[...], preferred_element_type=jnp.float32)
    y1_ref[...] = dinv * xw


def _tc2_body(t1_ref, y1_ref, dinv_ref, b1_ref, w2_ref, y2_ref):
    dinv = dinv_ref[...]
    h1 = jax.nn.relu(dinv * (t1_ref[0] + t1_ref[1] + y1_ref[...]) + b1_ref[...])
    y2_ref[...] = dinv * jnp.dot(h1, w2_ref[...], preferred_element_type=jnp.float32)


def _tc3_body(t2_ref, y2_ref, dinv_ref, b2_ref, batch_ref, out_ref, *, g):
    dinv = dinv_ref[...]
    h2 = jax.nn.relu(dinv * (t2_ref[0] + t2_ref[1] + y2_ref[...]) + b2_ref[...])
    n = h2.shape[0]
    seg = lax.broadcasted_iota(jnp.int32, (g, n), 0)
    mask = (seg == batch_ref[...]).astype(jnp.float32)
    sums = jnp.dot(mask, h2, preferred_element_type=jnp.float32)
    cnt = jnp.sum(mask, axis=1, keepdims=True)
    out_ref[...] = sums / jnp.maximum(cnt, 1.0)


def kernel(x, edge_index, batch, W1, b1, W2, b2):
    n, _ = x.shape
    e = edge_index.shape[1]
    d = W1.shape[1]
    g = 64

    ew = e // _NW
    nch = ew // _CH
    er = edge_index.reshape(2, _NW, nch, _CH)

    z16 = jnp.zeros((n, 16), jnp.float32)
    zd = jnp.zeros((n, d), jnp.float32)

    degp = _make_deg_kernel(n, nch)(er, z16)

    dinv, y1 = pl.pallas_call(
        _tc1_body,
        out_shape=[
            jax.ShapeDtypeStruct((n, 1), jnp.float32),
            jax.ShapeDtypeStruct((n, d), jnp.float32),
        ],
    )(x, W1, degp)

    mp = _make_mp_kernel(n, d, nch)
    t1 = mp(y1, er, zd)

    y2 = pl.pallas_call(
        _tc2_body,
        out_shape=jax.ShapeDtypeStruct((n, d), jnp.float32),
    )(t1, y1, dinv, b1.reshape(1, d), W2)

    t2 = mp(y2, er, zd)

    out = pl.pallas_call(
        functools.partial(_tc3_body, g=g),
        out_shape=jax.ShapeDtypeStruct((g, d), jnp.float32),
    )(t2, y2, dinv, b2.reshape(1, d), batch.reshape(1, n).astype(jnp.int32))
    return out
